# Initial kernel scaffold; baseline (speedup 1.0000x reference)
import jax
import jax.numpy as jnp
from jax.experimental import pallas as pl


def kernel(features_1, features_2, edge_index_1, edge_index_2, batch_1, batch_2, gcn_W0, gcn_b0, mi_W0, mi_b0, mo_W0, mo_b0, gcn_W1, gcn_b1, mi_W1, mi_b1, mo_W1, mo_b1, gcn_W2, gcn_b2, mi_W2, mi_b2, mo_W2, mo_b2, cs_W0, cs_b0, cs_W1, cs_b1, sc_W0, sc_b0, sc_W1, sc_b1):
    raise NotImplementedError("write your pallas kernel here")


# speedup vs baseline: 9.5639x; 9.5639x over previous
import functools

import jax
import jax.numpy as jnp
from jax import lax
from jax.experimental import pallas as pl
from jax.experimental.pallas import tpu as pltpu
from jax.experimental.pallas import tpu_sc as plsc

_N = 10000
_E = 320000
_B = 128
_NPAD = 10112
_CH = 128
_CPI = 8
_TILES = 16
_ITERS = 20
_CHUNKS = _CPI * _ITERS
_EPAD = _TILES * _CHUNKS * _CH
_ROWS = _NPAD // _TILES
_ERB = _EPAD // _CH
_NB = _NPAD // 128


def _sc_degree(dst2, ones16, zeros16):
    mesh = plsc.VectorSubcoreMesh(core_axis_name="c", subcore_axis_name="s")

    @functools.partial(
        pl.kernel, mesh=mesh,
        compiler_params=pltpu.CompilerParams(use_tc_tiling_on_sc=False),
        out_type=jax.ShapeDtypeStruct((2 * _NPAD, 16), jnp.float32),
        scratch_types=[
            pltpu.VMEM((_CPI, _CH), jnp.int32),
            pltpu.VMEM((_CH, 16), jnp.float32),
            pltpu.VMEM_SHARED((_NPAD, 16), jnp.float32),
        ],
    )
    def deg_kernel(dst_hbm, ones_hbm, z_hbm, out_hbm, didx, ones_v, acc):
        c = lax.axis_index("c")
        s = lax.axis_index("s")
        r0 = s * _ROWS
        pltpu.sync_copy(z_hbm.at[pl.ds(r0, _ROWS)], acc.at[pl.ds(r0, _ROWS)])
        pltpu.sync_copy(ones_hbm, ones_v)
        plsc.subcore_barrier()
        rbase = c * _ERB + s * _CHUNKS

        def body(i, carry):
            pltpu.sync_copy(dst_hbm.at[pl.ds(rbase + i * _CPI, _CPI)], didx)
            for j in range(_CPI):
                pltpu.sync_copy(ones_v, acc.at[didx.at[j]], add=True)
            return carry

        lax.fori_loop(0, _ITERS, body, 0)
        plsc.subcore_barrier()
        pltpu.sync_copy(acc.at[pl.ds(r0, _ROWS)],
                        out_hbm.at[pl.ds(c * _NPAD + r0, _ROWS)])

    return deg_kernel(dst2, ones16, zeros16)


def _sc_edge_scatter(y2, src2, dst2, zeros64):
    mesh = plsc.VectorSubcoreMesh(core_axis_name="c", subcore_axis_name="s")

    @functools.partial(
        pl.kernel, mesh=mesh,
        compiler_params=pltpu.CompilerParams(use_tc_tiling_on_sc=False),
        out_type=jax.ShapeDtypeStruct((2 * _NPAD, 64), jnp.float32),
        scratch_types=[
            pltpu.VMEM((_CPI, _CH), jnp.int32),
            pltpu.VMEM((_CPI, _CH), jnp.int32),
            pltpu.VMEM((_CPI * _CH, 64), jnp.float32),
            pltpu.VMEM_SHARED((_NPAD, 64), jnp.float32),
            pltpu.SemaphoreType.DMA,
        ],
    )
    def edge_kernel(y_hbm, src_hbm, dst_hbm, z_hbm, out_hbm,
                    sidx, didx, rows, acc, sem):
        c = lax.axis_index("c")
        s = lax.axis_index("s")
        r0 = s * _ROWS
        pltpu.sync_copy(z_hbm.at[pl.ds(r0, _ROWS)], acc.at[pl.ds(r0, _ROWS)])
        plsc.subcore_barrier()
        rbase = c * _ERB + s * _CHUNKS

        def body(i, carry):
            pltpu.sync_copy(src_hbm.at[pl.ds(rbase + i * _CPI, _CPI)], sidx)
            pltpu.sync_copy(dst_hbm.at[pl.ds(rbase + i * _CPI, _CPI)], didx)
            cps = [
                pltpu.async_copy(y_hbm.at[sidx.at[j]],
                                 rows.at[pl.ds(j * _CH, _CH)], sem)
                for j in range(_CPI)
            ]
            for cp in cps:
                cp.wait()
            for j in range(_CPI):
                pltpu.sync_copy(rows.at[pl.ds(j * _CH, _CH)],
                                acc.at[didx.at[j]], add=True)
            return carry

        lax.fori_loop(0, _ITERS, body, 0)
        plsc.subcore_barrier()
        pltpu.sync_copy(acc.at[pl.ds(r0, _ROWS)],
                        out_hbm.at[pl.ds(c * _NPAD + r0, _ROWS)])

    return edge_kernel(y2, src2, dst2, zeros64)


def _dinv_block(deg_ref, n):
    degc = jnp.max(deg_ref[0], axis=1, keepdims=True)
    row = n * 128 + lax.broadcasted_iota(jnp.int32, (128, 1), 0)
    return jnp.where(row < _N, 1.0 / jnp.sqrt(degc + 1.0), 0.0)


def _k1_body(x_ref, w_ref, deg_ref, y_ref):
    dinv = _dinv_block(deg_ref, pl.program_id(1))
    xw = jnp.dot(x_ref[0], w_ref[...], preferred_element_type=jnp.float32, precision=lax.Precision.HIGHEST)
    y_ref[0] = xw * dinv


def _tc_scale_matmul(x2, w, deg2):
    fin = w.shape[0]
    return pl.pallas_call(
        _k1_body,
        grid=(2, _NB),
        in_specs=[
            pl.BlockSpec((1, 128, fin), lambda g, n: (g, n, 0)),
            pl.BlockSpec((fin, 64), lambda g, n: (0, 0)),
            pl.BlockSpec((1, 128, 16), lambda g, n: (g, n, 0)),
        ],
        out_specs=pl.BlockSpec((1, 128, 64), lambda g, n: (g, n, 0)),
        out_shape=jax.ShapeDtypeStruct((2, _NPAD, 64), jnp.float32),
    )(x2, w, deg2)


def _k2_body(acc_ref, y_ref, deg_ref, b16_ref, gb_ref, miw_ref, mib_ref,
             h_ref, p_ref):
    n = pl.program_id(1)
    dinv = _dinv_block(deg_ref, n)
    h = jnp.maximum(dinv * (acc_ref[0] + y_ref[0]) + gb_ref[...], 0.0)
    h_ref[0] = h
    d = jnp.maximum(
        jnp.dot(h, miw_ref[...], preferred_element_type=jnp.float32, precision=lax.Precision.HIGHEST)
        + mib_ref[...], 0.0)
    bc = jnp.max(b16_ref[0], axis=1, keepdims=True)
    oneh = (bc == lax.broadcasted_iota(jnp.int32, (128, _B), 1).astype(jnp.float32))
    pp = lax.dot_general(oneh.astype(jnp.float32), d,
                         (((0,), (0,)), ((), ())),
                         preferred_element_type=jnp.float32,
                         precision=lax.Precision.HIGHEST)

    @pl.when(n == 0)
    def _():
        p_ref[0] = pp

    @pl.when(n != 0)
    def _():
        p_ref[0] = p_ref[0] + pp


def _tc_update_pool(acc2, y2, deg2, b16, gcn_b, mi_w, mi_b):
    return pl.pallas_call(
        _k2_body,
        grid=(2, _NB),
        in_specs=[
            pl.BlockSpec((1, 128, 64), lambda g, n: (g, n, 0)),
            pl.BlockSpec((1, 128, 64), lambda g, n: (g, n, 0)),
            pl.BlockSpec((1, 128, 16), lambda g, n: (g, n, 0)),
            pl.BlockSpec((1, 128, 16), lambda g, n: (g, n, 0)),
            pl.BlockSpec((1, 64), lambda g, n: (0, 0)),
            pl.BlockSpec((64, 64), lambda g, n: (0, 0)),
            pl.BlockSpec((1, 64), lambda g, n: (0, 0)),
        ],
        out_specs=[
            pl.BlockSpec((1, 128, 64), lambda g, n: (g, n, 0)),
            pl.BlockSpec((1, _B, 64), lambda g, n: (g, 0, 0)),
        ],
        out_shape=[
            jax.ShapeDtypeStruct((2, _NPAD, 64), jnp.float32),
            jax.ShapeDtypeStruct((2, _B, 64), jnp.float32),
        ],
    )(acc2, y2, deg2, b16, gcn_b, mi_w, mi_b)


def _k3_body(p0_ref, p1_ref, p2_ref,
             mow0_ref, mob0_ref, mow1_ref, mob1_ref, mow2_ref, mob2_ref,
             csw0a_ref, csw0b_ref, csw0c_ref, csb0_ref,
             csw1_ref, csb1_ref, scw0_ref, scb0_ref, scw1_ref, scb1_ref,
             out_ref):
    def diff(p_ref, mow_ref, mob_ref):
        o1 = jnp.maximum(
            jnp.dot(p_ref[0], mow_ref[...],
                    preferred_element_type=jnp.float32, precision=lax.Precision.HIGHEST) + mob_ref[...], 0.0)
        o2 = jnp.maximum(
            jnp.dot(p_ref[1], mow_ref[...],
                    preferred_element_type=jnp.float32, precision=lax.Precision.HIGHEST) + mob_ref[...], 0.0)
        return jnp.exp(-jnp.square(o1 - o2))

    d0 = diff(p0_ref, mow0_ref, mob0_ref)
    d1 = diff(p1_ref, mow1_ref, mob1_ref)
    d2 = diff(p2_ref, mow2_ref, mob2_ref)
    h = (jnp.dot(d0, csw0a_ref[...], preferred_element_type=jnp.float32, precision=lax.Precision.HIGHEST)
         + jnp.dot(d1, csw0b_ref[...], preferred_element_type=jnp.float32, precision=lax.Precision.HIGHEST)
         + jnp.dot(d2, csw0c_ref[...], preferred_element_type=jnp.float32, precision=lax.Precision.HIGHEST)
         + csb0_ref[...])
    h = jnp.maximum(h, 0.0)
    h = jnp.tanh(
        jnp.dot(h, csw1_ref[...], preferred_element_type=jnp.float32, precision=lax.Precision.HIGHEST)
        + csb1_ref[...])
    s = jnp.maximum(
        jnp.dot(h, scw0_ref[...], preferred_element_type=jnp.float32, precision=lax.Precision.HIGHEST)
        + scb0_ref[...], 0.0)
    z = (jnp.dot(s, scw1_ref[...], preferred_element_type=jnp.float32, precision=lax.Precision.HIGHEST)
         + scb1_ref[...])
    out_ref[...] = 1.0 / (1.0 + jnp.exp(-z))


def _tc_head(pooled, mo, cs_w0, cs_b0, cs_w1, cs_b1,
             sc_w0, sc_b0, sc_w1, sc_b1):
    args = [pooled[0], pooled[1], pooled[2],
            mo[0][0], mo[0][1].reshape(1, -1),
            mo[1][0], mo[1][1].reshape(1, -1),
            mo[2][0], mo[2][1].reshape(1, -1),
            cs_w0[0:64], cs_w0[64:128], cs_w0[128:192],
            cs_b0.reshape(1, -1),
            cs_w1, cs_b1.reshape(1, -1),
            sc_w0, sc_b0.reshape(1, -1),
            sc_w1, sc_b1.reshape(1, 1)]
    return pl.pallas_call(
        _k3_body,
        out_shape=jax.ShapeDtypeStruct((_B, 1), jnp.float32),
    )(*args)


def kernel(features_1, features_2, edge_index_1, edge_index_2,
           batch_1, batch_2,
           gcn_W0, gcn_b0, mi_W0, mi_b0, mo_W0, mo_b0,
           gcn_W1, gcn_b1, mi_W1, mi_b1, mo_W1, mo_b1,
           gcn_W2, gcn_b2, mi_W2, mi_b2, mo_W2, mo_b2,
           cs_W0, cs_b0, cs_W1, cs_b1, sc_W0, sc_b0, sc_W1, sc_b1):
    f32 = jnp.float32
    epad = jnp.full((_EPAD - _E,), _N, jnp.int32)
    src2 = jnp.concatenate([
        edge_index_1[0], epad,
        edge_index_2[0] + _NPAD, epad + _NPAD,
    ]).reshape(2 * _ERB, _CH)
    dst2 = jnp.concatenate([
        edge_index_1[1], epad, edge_index_2[1], epad,
    ]).reshape(2 * _ERB, _CH)

    zeros16 = jnp.zeros((_NPAD, 16), f32)
    zeros64 = jnp.zeros((_NPAD, 64), f32)
    ones16 = jnp.ones((_CH, 16), f32)

    deg2 = _sc_degree(dst2, ones16, zeros16).reshape(2, _NPAD, 16)

    bpad = jnp.full((_NPAD - _N,), _B, jnp.int32)
    b16 = jnp.broadcast_to(
        jnp.stack([jnp.concatenate([batch_1, bpad]),
                   jnp.concatenate([batch_2, bpad])]).astype(f32)[..., None],
        (2, _NPAD, 16))

    x = jnp.pad(jnp.stack([features_1, features_2]),
                ((0, 0), (0, _NPAD - _N), (0, 0)))

    gcn = [(gcn_W0, gcn_b0.reshape(1, -1)),
           (gcn_W1, gcn_b1.reshape(1, -1)),
           (gcn_W2, gcn_b2.reshape(1, -1))]
    mi = [(mi_W0, mi_b0.reshape(1, -1)),
          (mi_W1, mi_b1.reshape(1, -1)),
          (mi_W2, mi_b2.reshape(1, -1))]
    mo = [(mo_W0, mo_b0), (mo_W1, mo_b1), (mo_W2, mo_b2)]

    pooled = []
    for i in range(3):
        y = _tc_scale_matmul(x, gcn[i][0], deg2)
        acc = _sc_edge_scatter(y.reshape(2 * _NPAD, 64), src2, dst2,
                               zeros64).reshape(2, _NPAD, 64)
        x, p = _tc_update_pool(acc, y, deg2, b16,
                               gcn[i][1], mi[i][0], mi[i][1])
        pooled.append(p)

    score = _tc_head(pooled, mo, cs_W0, cs_b0, cs_W1, cs_b1,
                     sc_W0, sc_b0, sc_W1, sc_b1)
    return score.reshape(-1)


# speedup vs baseline: 9.9706x; 1.0425x over previous
import functools

import jax
import jax.numpy as jnp
from jax import lax
from jax.experimental import pallas as pl
from jax.experimental.pallas import tpu as pltpu
from jax.experimental.pallas import tpu_sc as plsc

_N = 10000
_E = 320000
_B = 128
_NPAD = 10112
_CH = 128
_CPI = 8
_TILES = 16
_ITERS = 20
_CHUNKS = _CPI * _ITERS
_PCPI = 4
_PH = _CHUNKS // _PCPI
_EPAD = _TILES * _CHUNKS * _CH
_ROWS = _NPAD // _TILES
_ERB = _EPAD // _CH
_NB = _NPAD // 128


def _sc_degree(dst2, ones16, zeros16):
    mesh = plsc.VectorSubcoreMesh(core_axis_name="c", subcore_axis_name="s")

    @functools.partial(
        pl.kernel, mesh=mesh,
        compiler_params=pltpu.CompilerParams(use_tc_tiling_on_sc=False),
        out_type=jax.ShapeDtypeStruct((2 * _NPAD, 16), jnp.float32),
        scratch_types=[
            pltpu.VMEM((_CPI, _CH), jnp.int32),
            pltpu.VMEM((_CH, 16), jnp.float32),
            pltpu.VMEM_SHARED((_NPAD, 16), jnp.float32),
        ],
    )
    def deg_kernel(dst_hbm, ones_hbm, z_hbm, out_hbm, didx, ones_v, acc):
        c = lax.axis_index("c")
        s = lax.axis_index("s")
        r0 = s * _ROWS
        pltpu.sync_copy(z_hbm.at[pl.ds(r0, _ROWS)], acc.at[pl.ds(r0, _ROWS)])
        pltpu.sync_copy(ones_hbm, ones_v)
        plsc.subcore_barrier()
        rbase = c * _ERB + s * _CHUNKS

        def body(i, carry):
            pltpu.sync_copy(dst_hbm.at[pl.ds(rbase + i * _CPI, _CPI)], didx)
            for j in range(_CPI):
                pltpu.sync_copy(ones_v, acc.at[didx.at[j]], add=True)
            return carry

        lax.fori_loop(0, _ITERS, body, 0)
        plsc.subcore_barrier()
        pltpu.sync_copy(acc.at[pl.ds(r0, _ROWS)],
                        out_hbm.at[pl.ds(c * _NPAD + r0, _ROWS)])

    return deg_kernel(dst2, ones16, zeros16)


def _sc_edge_scatter(y2, src2, dst2, zeros64):
    mesh = plsc.VectorSubcoreMesh(core_axis_name="c", subcore_axis_name="s")

    @functools.partial(
        pl.kernel, mesh=mesh,
        compiler_params=pltpu.CompilerParams(use_tc_tiling_on_sc=False),
        out_type=jax.ShapeDtypeStruct((2 * _NPAD, 64), jnp.float32),
        scratch_types=[
            pltpu.VMEM((2, _PCPI, _CH), jnp.int32),
            pltpu.VMEM((2, _PCPI, _CH), jnp.int32),
            pltpu.VMEM((2, _PCPI * _CH, 64), jnp.float32),
            pltpu.VMEM_SHARED((_NPAD, 64), jnp.float32),
            pltpu.SemaphoreType.DMA,
            pltpu.SemaphoreType.DMA,
        ],
    )
    def edge_kernel(y_hbm, src_hbm, dst_hbm, z_hbm, out_hbm,
                    sidx, didx, rows, acc, sem_g, sem_s):
        c = lax.axis_index("c")
        s = lax.axis_index("s")
        r0 = s * _ROWS
        pltpu.sync_copy(z_hbm.at[pl.ds(r0, _ROWS)], acc.at[pl.ds(r0, _ROWS)])
        plsc.subcore_barrier()
        rbase = c * _ERB + s * _CHUNKS

        def load_idx(p, b):
            off = rbase + p * _PCPI
            pltpu.sync_copy(src_hbm.at[pl.ds(off, _PCPI)], sidx.at[b])
            pltpu.sync_copy(dst_hbm.at[pl.ds(off, _PCPI)], didx.at[b])

        def gathers(b):
            return [pltpu.make_async_copy(
                        y_hbm.at[sidx.at[b, j]],
                        rows.at[b, pl.ds(j * _CH, _CH)], sem_g)
                    for j in range(_PCPI)]

        def scatters(b):
            return [pltpu.make_async_copy(
                        rows.at[b, pl.ds(j * _CH, _CH)],
                        acc.at[didx.at[b, j]], sem_s)
                    for j in range(_PCPI)]

        load_idx(0, 0)
        for cp in gathers(0):
            cp.start()

        def phase(p, b):
            for cp in gathers(b):
                cp.wait()
            for cp in scatters(b):
                cp.start(add=True)

            @pl.when(p + 1 < _PH)
            def _():
                @pl.when(p >= 1)
                def _():
                    for cp in scatters(1 - b):
                        cp.wait()
                load_idx(p + 1, 1 - b)
                for cp in gathers(1 - b):
                    cp.start()

        @pl.loop(0, _PH, step=2)
        def _(k):
            phase(k, 0)
            phase(k + 1, 1)

        for cp in scatters(0):
            cp.wait()
        for cp in scatters(1):
            cp.wait()
        plsc.subcore_barrier()
        pltpu.sync_copy(acc.at[pl.ds(r0, _ROWS)],
                        out_hbm.at[pl.ds(c * _NPAD + r0, _ROWS)])

    return edge_kernel(y2, src2, dst2, zeros64)


def _dinv_block(deg_ref, n):
    degc = jnp.max(deg_ref[0], axis=1, keepdims=True)
    row = n * 128 + lax.broadcasted_iota(jnp.int32, (128, 1), 0)
    return jnp.where(row < _N, 1.0 / jnp.sqrt(degc + 1.0), 0.0)


def _k1_body(x_ref, w_ref, deg_ref, y_ref):
    dinv = _dinv_block(deg_ref, pl.program_id(1))
    xw = jnp.dot(x_ref[0], w_ref[...], preferred_element_type=jnp.float32, precision=lax.Precision.HIGHEST)
    y_ref[0] = xw * dinv


def _tc_scale_matmul(x2, w, deg2):
    fin = w.shape[0]
    return pl.pallas_call(
        _k1_body,
        grid=(2, _NB),
        in_specs=[
            pl.BlockSpec((1, 128, fin), lambda g, n: (g, n, 0)),
            pl.BlockSpec((fin, 64), lambda g, n: (0, 0)),
            pl.BlockSpec((1, 128, 16), lambda g, n: (g, n, 0)),
        ],
        out_specs=pl.BlockSpec((1, 128, 64), lambda g, n: (g, n, 0)),
        out_shape=jax.ShapeDtypeStruct((2, _NPAD, 64), jnp.float32),
    )(x2, w, deg2)


def _k2_body(acc_ref, y_ref, deg_ref, b16_ref, gb_ref, miw_ref, mib_ref,
             h_ref, p_ref):
    n = pl.program_id(1)
    dinv = _dinv_block(deg_ref, n)
    h = jnp.maximum(dinv * (acc_ref[0] + y_ref[0]) + gb_ref[...], 0.0)
    h_ref[0] = h
    d = jnp.maximum(
        jnp.dot(h, miw_ref[...], preferred_element_type=jnp.float32, precision=lax.Precision.HIGHEST)
        + mib_ref[...], 0.0)
    bc = jnp.max(b16_ref[0], axis=1, keepdims=True)
    oneh = (bc == lax.broadcasted_iota(jnp.int32, (128, _B), 1).astype(jnp.float32))
    pp = lax.dot_general(oneh.astype(jnp.float32), d,
                         (((0,), (0,)), ((), ())),
                         preferred_element_type=jnp.float32,
                         precision=lax.Precision.HIGHEST)

    @pl.when(n == 0)
    def _():
        p_ref[0] = pp

    @pl.when(n != 0)
    def _():
        p_ref[0] = p_ref[0] + pp


def _tc_update_pool(acc2, y2, deg2, b16, gcn_b, mi_w, mi_b):
    return pl.pallas_call(
        _k2_body,
        grid=(2, _NB),
        in_specs=[
            pl.BlockSpec((1, 128, 64), lambda g, n: (g, n, 0)),
            pl.BlockSpec((1, 128, 64), lambda g, n: (g, n, 0)),
            pl.BlockSpec((1, 128, 16), lambda g, n: (g, n, 0)),
            pl.BlockSpec((1, 128, 16), lambda g, n: (g, n, 0)),
            pl.BlockSpec((1, 64), lambda g, n: (0, 0)),
            pl.BlockSpec((64, 64), lambda g, n: (0, 0)),
            pl.BlockSpec((1, 64), lambda g, n: (0, 0)),
        ],
        out_specs=[
            pl.BlockSpec((1, 128, 64), lambda g, n: (g, n, 0)),
            pl.BlockSpec((1, _B, 64), lambda g, n: (g, 0, 0)),
        ],
        out_shape=[
            jax.ShapeDtypeStruct((2, _NPAD, 64), jnp.float32),
            jax.ShapeDtypeStruct((2, _B, 64), jnp.float32),
        ],
    )(acc2, y2, deg2, b16, gcn_b, mi_w, mi_b)


def _k3_body(p0_ref, p1_ref, p2_ref,
             mow0_ref, mob0_ref, mow1_ref, mob1_ref, mow2_ref, mob2_ref,
             csw0a_ref, csw0b_ref, csw0c_ref, csb0_ref,
             csw1_ref, csb1_ref, scw0_ref, scb0_ref, scw1_ref, scb1_ref,
             out_ref):
    def diff(p_ref, mow_ref, mob_ref):
        o1 = jnp.maximum(
            jnp.dot(p_ref[0], mow_ref[...],
                    preferred_element_type=jnp.float32, precision=lax.Precision.HIGHEST) + mob_ref[...], 0.0)
        o2 = jnp.maximum(
            jnp.dot(p_ref[1], mow_ref[...],
                    preferred_element_type=jnp.float32, precision=lax.Precision.HIGHEST) + mob_ref[...], 0.0)
        return jnp.exp(-jnp.square(o1 - o2))

    d0 = diff(p0_ref, mow0_ref, mob0_ref)
    d1 = diff(p1_ref, mow1_ref, mob1_ref)
    d2 = diff(p2_ref, mow2_ref, mob2_ref)
    h = (jnp.dot(d0, csw0a_ref[...], preferred_element_type=jnp.float32, precision=lax.Precision.HIGHEST)
         + jnp.dot(d1, csw0b_ref[...], preferred_element_type=jnp.float32, precision=lax.Precision.HIGHEST)
         + jnp.dot(d2, csw0c_ref[...], preferred_element_type=jnp.float32, precision=lax.Precision.HIGHEST)
         + csb0_ref[...])
    h = jnp.maximum(h, 0.0)
    h = jnp.tanh(
        jnp.dot(h, csw1_ref[...], preferred_element_type=jnp.float32, precision=lax.Precision.HIGHEST)
        + csb1_ref[...])
    s = jnp.maximum(
        jnp.dot(h, scw0_ref[...], preferred_element_type=jnp.float32, precision=lax.Precision.HIGHEST)
        + scb0_ref[...], 0.0)
    z = (jnp.dot(s, scw1_ref[...], preferred_element_type=jnp.float32, precision=lax.Precision.HIGHEST)
         + scb1_ref[...])
    out_ref[...] = 1.0 / (1.0 + jnp.exp(-z))


def _tc_head(pooled, mo, cs_w0, cs_b0, cs_w1, cs_b1,
             sc_w0, sc_b0, sc_w1, sc_b1):
    args = [pooled[0], pooled[1], pooled[2],
            mo[0][0], mo[0][1].reshape(1, -1),
            mo[1][0], mo[1][1].reshape(1, -1),
            mo[2][0], mo[2][1].reshape(1, -1),
            cs_w0[0:64], cs_w0[64:128], cs_w0[128:192],
            cs_b0.reshape(1, -1),
            cs_w1, cs_b1.reshape(1, -1),
            sc_w0, sc_b0.reshape(1, -1),
            sc_w1, sc_b1.reshape(1, 1)]
    return pl.pallas_call(
        _k3_body,
        out_shape=jax.ShapeDtypeStruct((_B, 1), jnp.float32),
    )(*args)


def kernel(features_1, features_2, edge_index_1, edge_index_2,
           batch_1, batch_2,
           gcn_W0, gcn_b0, mi_W0, mi_b0, mo_W0, mo_b0,
           gcn_W1, gcn_b1, mi_W1, mi_b1, mo_W1, mo_b1,
           gcn_W2, gcn_b2, mi_W2, mi_b2, mo_W2, mo_b2,
           cs_W0, cs_b0, cs_W1, cs_b1, sc_W0, sc_b0, sc_W1, sc_b1):
    f32 = jnp.float32
    epad = jnp.full((_EPAD - _E,), _N, jnp.int32)
    src2 = jnp.concatenate([
        edge_index_1[0], epad,
        edge_index_2[0] + _NPAD, epad + _NPAD,
    ]).reshape(2 * _ERB, _CH)
    dst2 = jnp.concatenate([
        edge_index_1[1], epad, edge_index_2[1], epad,
    ]).reshape(2 * _ERB, _CH)

    zeros16 = jnp.zeros((_NPAD, 16), f32)
    zeros64 = jnp.zeros((_NPAD, 64), f32)
    ones16 = jnp.ones((_CH, 16), f32)

    deg2 = _sc_degree(dst2, ones16, zeros16).reshape(2, _NPAD, 16)

    bpad = jnp.full((_NPAD - _N,), _B, jnp.int32)
    b16 = jnp.broadcast_to(
        jnp.stack([jnp.concatenate([batch_1, bpad]),
                   jnp.concatenate([batch_2, bpad])]).astype(f32)[..., None],
        (2, _NPAD, 16))

    x = jnp.pad(jnp.stack([features_1, features_2]),
                ((0, 0), (0, _NPAD - _N), (0, 0)))

    gcn = [(gcn_W0, gcn_b0.reshape(1, -1)),
           (gcn_W1, gcn_b1.reshape(1, -1)),
           (gcn_W2, gcn_b2.reshape(1, -1))]
    mi = [(mi_W0, mi_b0.reshape(1, -1)),
          (mi_W1, mi_b1.reshape(1, -1)),
          (mi_W2, mi_b2.reshape(1, -1))]
    mo = [(mo_W0, mo_b0), (mo_W1, mo_b1), (mo_W2, mo_b2)]

    pooled = []
    for i in range(3):
        y = _tc_scale_matmul(x, gcn[i][0], deg2)
        acc = _sc_edge_scatter(y.reshape(2 * _NPAD, 64), src2, dst2,
                               zeros64).reshape(2, _NPAD, 64)
        x, p = _tc_update_pool(acc, y, deg2, b16,
                               gcn[i][1], mi[i][0], mi[i][1])
        pooled.append(p)

    score = _tc_head(pooled, mo, cs_W0, cs_b0, cs_W1, cs_b1,
                     sc_W0, sc_b0, sc_W1, sc_b1)
    return score.reshape(-1)
